# baseline (speedup 1.0000x reference)
import functools

import jax
import jax.numpy as jnp
from jax import lax
from jax.experimental import pallas as pl
from jax.experimental.pallas import tpu as pltpu
from jax.experimental.pallas import tpu_sc as plsc

_E = 64
_K = 8
_TB = 1024
_NC = 2
_NS = 16
_NW = _NC * _NS
_L = 16


def _mm_block(x_ref, w_ref, b_ref, bt_ref, logits_ref, probs_t_ref):
    x = x_ref[...]
    w = w_ref[...]
    l = jnp.dot(x, w, preferred_element_type=jnp.float32) + b_ref[...]
    logits_ref[...] = l
    lt = jax.lax.dot_general(
        w, x, (((0,), (1,)), ((), ())),
        preferred_element_type=jnp.float32,
    ) + bt_ref[...]
    m = jnp.max(lt, axis=0, keepdims=True)
    e = jnp.exp(lt - m)
    probs_t_ref[...] = e / jnp.sum(e, axis=0, keepdims=True)


def _matmul_probs(x2d, W, b2d, bt2d, chunk, nchunks):
    T, D = x2d.shape
    tch = T // nchunks
    nb = tch // _TB
    off = chunk * nb
    return pl.pallas_call(
        _mm_block,
        grid=(nb,),
        in_specs=[
            pl.BlockSpec((_TB, D), lambda i: (off + i, 0)),
            pl.BlockSpec((D, _E), lambda i: (0, 0)),
            pl.BlockSpec((1, _E), lambda i: (0, 0)),
            pl.BlockSpec((_E, 1), lambda i: (0, 0)),
        ],
        out_specs=[
            pl.BlockSpec((_TB, _E), lambda i: (i, 0)),
            pl.BlockSpec((_E, _TB), lambda i: (0, i)),
        ],
        out_shape=[
            jax.ShapeDtypeStruct((tch, _E), jnp.float32),
            jax.ShapeDtypeStruct((_E, tch), jnp.float32),
        ],
        compiler_params=pltpu.CompilerParams(
            dimension_semantics=("arbitrary",),
        ),
    )(x2d, W, b2d, bt2d)


def _sc_route_body(tpw, probs_t_hbm, gates_t_hbm, idx_t_hbm, p_v, g_v, i_v):
    wid = lax.axis_index("s") * _NC + lax.axis_index("c")
    base = wid * tpw
    pltpu.sync_copy(probs_t_hbm.at[:, pl.ds(base, tpw)], p_v)

    def group(g, carry):
        off = g * _L
        himask = jnp.full((_L,), 0xFFFFFFC0, jnp.uint32)
        s = [jnp.zeros((_L,), jnp.uint32) for _ in range(_K)]
        for e in range(_E):
            cv = p_v[e, pl.ds(off, _L)]
            cu = lax.bitcast_convert_type(cv, jnp.uint32)
            c = (cu & himask) | jnp.uint32(63 - e)
            for i in range(_K):
                si = s[i]
                s[i] = jnp.maximum(c, si)
                c = jnp.minimum(c, si)
        vals = [lax.bitcast_convert_type(s[i], jnp.float32) for i in range(_K)]
        ids = [
            (jnp.int32(63) - (s[i] & jnp.uint32(63)).astype(jnp.int32))
            for i in range(_K)
        ]
        tot = vals[0]
        for i in range(1, _K):
            tot = tot + vals[i]
        tot = tot + 1e-9
        for i in range(_K):
            g_v[i, pl.ds(off, _L)] = vals[i] / tot
            i_v[i, pl.ds(off, _L)] = ids[i]
        return carry

    lax.fori_loop(0, tpw // _L, group, 0)
    pltpu.sync_copy(g_v, gates_t_hbm.at[:, pl.ds(base, tpw)])
    pltpu.sync_copy(i_v, idx_t_hbm.at[:, pl.ds(base, tpw)])


def _sc_route(probs_t):
    T = probs_t.shape[1]
    tpw = T // _NW
    mesh = plsc.VectorSubcoreMesh(core_axis_name="c", subcore_axis_name="s")
    f = functools.partial(
        pl.kernel,
        mesh=mesh,
        out_type=[
            jax.ShapeDtypeStruct((_K, T), jnp.float32),
            jax.ShapeDtypeStruct((_K, T), jnp.int32),
        ],
        scratch_types=[
            pltpu.VMEM((_E, tpw), jnp.float32),
            pltpu.VMEM((_K, tpw), jnp.float32),
            pltpu.VMEM((_K, tpw), jnp.int32),
        ],
    )(functools.partial(_sc_route_body, tpw))
    return f(probs_t)


_TBT = 4096


def _tr_block(g_ref, i_ref, go_ref, io_ref):
    go_ref[...] = g_ref[...].T
    io_ref[...] = i_ref[...].T


def _transpose_out(gates_t, idx_t):
    T = gates_t.shape[1]
    return pl.pallas_call(
        _tr_block,
        grid=(T // _TBT,),
        in_specs=[
            pl.BlockSpec((_K, _TBT), lambda j: (0, j)),
            pl.BlockSpec((_K, _TBT), lambda j: (0, j)),
        ],
        out_specs=[
            pl.BlockSpec((_TBT, _K), lambda j: (j, 0)),
            pl.BlockSpec((_TBT, _K), lambda j: (j, 0)),
        ],
        out_shape=[
            jax.ShapeDtypeStruct((T, _K), jnp.float32),
            jax.ShapeDtypeStruct((T, _K), jnp.int32),
        ],
        compiler_params=pltpu.CompilerParams(
            dimension_semantics=("parallel",),
        ),
    )(gates_t, idx_t)


_NCHUNKS = 1


@jax.jit
def _router(x2d, W, b2d, bt2d):
    lg, gt, it = [], [], []
    for c in range(_NCHUNKS):
        logits_c, probs_t_c = _matmul_probs(x2d, W, b2d, bt2d, c, _NCHUNKS)
        gates_t_c, idx_t_c = _sc_route(probs_t_c)
        lg.append(logits_c)
        gt.append(gates_t_c)
        it.append(idx_t_c)
    logits = jnp.concatenate(lg, axis=0)
    gates_t = jnp.concatenate(gt, axis=1)
    idx_t = jnp.concatenate(it, axis=1)
    gates, idx = _transpose_out(gates_t, idx_t)
    return logits, gates, idx


def kernel(x, W, b):
    B, S, D = x.shape
    x2d = x.reshape(B * S, D)
    logits, gates, idx = _router(x2d, W, b.reshape(1, _E), b.reshape(_E, 1))
    return (
        gates.reshape(B, S, _K),
        idx.reshape(B, S, _K),
        logits.reshape(B, S, _E),
    )


# speedup vs baseline: 1.1927x; 1.1927x over previous
import functools

import jax
import jax.numpy as jnp
from jax import lax
from jax.experimental import pallas as pl
from jax.experimental.pallas import tpu as pltpu
from jax.experimental.pallas import tpu_sc as plsc

_E = 64
_K = 8
_TB = 1024
_NC = 2
_NS = 16
_NW = _NC * _NS
_L = 16


def _mm_block(x_ref, w_ref, b_ref, logits_ref, probs_t_ref):
    x = x_ref[...]
    w = w_ref[...]
    l = jnp.dot(x, w, preferred_element_type=jnp.float32) + b_ref[...]
    logits_ref[...] = l
    m = jnp.max(l, axis=1, keepdims=True)
    e = jnp.exp(l - m)
    p = e / jnp.sum(e, axis=1, keepdims=True)
    probs_t_ref[...] = p.T


def _matmul_probs(x2d, W, b2d, chunk, nchunks):
    T, D = x2d.shape
    tch = T // nchunks
    nb = tch // _TB
    off = chunk * nb
    return pl.pallas_call(
        _mm_block,
        grid=(nb,),
        in_specs=[
            pl.BlockSpec((_TB, D), lambda i: (off + i, 0)),
            pl.BlockSpec((D, _E), lambda i: (0, 0)),
            pl.BlockSpec((1, _E), lambda i: (0, 0)),
        ],
        out_specs=[
            pl.BlockSpec((_TB, _E), lambda i: (i, 0)),
            pl.BlockSpec((_E, _TB), lambda i: (0, i)),
        ],
        out_shape=[
            jax.ShapeDtypeStruct((tch, _E), jnp.float32),
            jax.ShapeDtypeStruct((_E, tch), jnp.float32),
        ],
        compiler_params=pltpu.CompilerParams(
            dimension_semantics=("arbitrary",),
        ),
    )(x2d, W, b2d)


def _sc_route_body(tpw, probs_t_hbm, gates_t_hbm, idx_t_hbm, p_v, g_v, i_v):
    wid = lax.axis_index("s") * _NC + lax.axis_index("c")
    base = wid * tpw
    pltpu.sync_copy(probs_t_hbm.at[:, pl.ds(base, tpw)], p_v)

    def group(g, carry):
        off = g * _L
        himask = jnp.full((_L,), 0xFFFFFFC0, jnp.uint32)
        s = [jnp.zeros((_L,), jnp.uint32) for _ in range(_K)]
        for e in range(_E):
            cv = p_v[e, pl.ds(off, _L)]
            cu = lax.bitcast_convert_type(cv, jnp.uint32)
            c = (cu & himask) | jnp.uint32(63 - e)
            for i in range(_K):
                si = s[i]
                s[i] = jnp.maximum(c, si)
                c = jnp.minimum(c, si)
        vals = [lax.bitcast_convert_type(s[i], jnp.float32) for i in range(_K)]
        ids = [
            (jnp.int32(63) - (s[i] & jnp.uint32(63)).astype(jnp.int32))
            for i in range(_K)
        ]
        tot = vals[0]
        for i in range(1, _K):
            tot = tot + vals[i]
        tot = tot + 1e-9
        for i in range(_K):
            g_v[i, pl.ds(off, _L)] = vals[i] / tot
            i_v[i, pl.ds(off, _L)] = ids[i]
        return carry

    lax.fori_loop(0, tpw // _L, group, 0)
    pltpu.sync_copy(g_v, gates_t_hbm.at[:, pl.ds(base, tpw)])
    pltpu.sync_copy(i_v, idx_t_hbm.at[:, pl.ds(base, tpw)])


def _sc_route(probs_t):
    T = probs_t.shape[1]
    tpw = T // _NW
    mesh = plsc.VectorSubcoreMesh(core_axis_name="c", subcore_axis_name="s")
    f = functools.partial(
        pl.kernel,
        mesh=mesh,
        out_type=[
            jax.ShapeDtypeStruct((_K, T), jnp.float32),
            jax.ShapeDtypeStruct((_K, T), jnp.int32),
        ],
        scratch_types=[
            pltpu.VMEM((_E, tpw), jnp.float32),
            pltpu.VMEM((_K, tpw), jnp.float32),
            pltpu.VMEM((_K, tpw), jnp.int32),
        ],
    )(functools.partial(_sc_route_body, tpw))
    return f(probs_t)


_NCHUNKS = 1


@jax.jit
def _router(x2d, W, b2d):
    lg, gt, it = [], [], []
    for c in range(_NCHUNKS):
        logits_c, probs_t_c = _matmul_probs(x2d, W, b2d, c, _NCHUNKS)
        gates_t_c, idx_t_c = _sc_route(probs_t_c)
        lg.append(logits_c)
        gt.append(gates_t_c)
        it.append(idx_t_c)
    logits = jnp.concatenate(lg, axis=0)
    gates = jnp.concatenate(gt, axis=1).T
    idx = jnp.concatenate(it, axis=1).T
    return logits, gates, idx


def kernel(x, W, b):
    B, S, D = x.shape
    x2d = x.reshape(B * S, D)
    logits, gates, idx = _router(x2d, W, b.reshape(1, _E))
    return (
        gates.reshape(B, S, _K),
        idx.reshape(B, S, _K),
        logits.reshape(B, S, _E),
    )


# speedup vs baseline: 1.1949x; 1.0018x over previous
import functools

import jax
import jax.numpy as jnp
from jax import lax
from jax.experimental import pallas as pl
from jax.experimental.pallas import tpu as pltpu
from jax.experimental.pallas import tpu_sc as plsc

_E = 64
_K = 8
_TB = 1024
_NC = 2
_NS = 16
_NW = _NC * _NS
_L = 16


def _mm_block(x_ref, w_ref, b_ref, logits_ref, probs_t_ref):
    x = x_ref[...]
    w = w_ref[...]
    l = jnp.dot(x, w, preferred_element_type=jnp.float32) + b_ref[...]
    logits_ref[...] = l
    m = jnp.max(l, axis=1, keepdims=True)
    e = jnp.exp(l - m)
    p = e / jnp.sum(e, axis=1, keepdims=True)
    probs_t_ref[...] = p.T


def _matmul_probs(x2d, W, b2d, chunk, nchunks):
    T, D = x2d.shape
    tch = T // nchunks
    nb = tch // _TB
    off = chunk * nb
    return pl.pallas_call(
        _mm_block,
        grid=(nb,),
        in_specs=[
            pl.BlockSpec((_TB, D), lambda i: (off + i, 0)),
            pl.BlockSpec((D, _E), lambda i: (0, 0)),
            pl.BlockSpec((1, _E), lambda i: (0, 0)),
        ],
        out_specs=[
            pl.BlockSpec((_TB, _E), lambda i: (i, 0)),
            pl.BlockSpec((_E, _TB), lambda i: (0, i)),
        ],
        out_shape=[
            jax.ShapeDtypeStruct((tch, _E), jnp.float32),
            jax.ShapeDtypeStruct((_E, tch), jnp.float32),
        ],
        compiler_params=pltpu.CompilerParams(
            dimension_semantics=("arbitrary",),
        ),
    )(x2d, W, b2d)


def _sc_route_body(tpw, probs_t_hbm, gates_t_hbm, idx_t_hbm, p_v, g_v, i_v):
    wid = lax.axis_index("s") * _NC + lax.axis_index("c")
    base = wid * tpw
    pltpu.sync_copy(probs_t_hbm.at[:, pl.ds(base, tpw)], p_v)

    def group(g, carry):
        off = g * _L
        himask = jnp.full((_L,), 0xFFFFFFC0, jnp.uint32)
        s = [jnp.zeros((_L,), jnp.uint32) for _ in range(_K)]
        for e in range(_E):
            cv = p_v[e, pl.ds(off, _L)]
            cu = lax.bitcast_convert_type(cv, jnp.uint32)
            c = (cu & himask) | jnp.uint32(63 - e)
            for i in range(_K):
                si = s[i]
                s[i] = jnp.maximum(c, si)
                c = jnp.minimum(c, si)
        vals = [lax.bitcast_convert_type(s[i], jnp.float32) for i in range(_K)]
        ids = [
            (jnp.int32(63) - (s[i] & jnp.uint32(63)).astype(jnp.int32))
            for i in range(_K)
        ]
        tot = vals[0]
        for i in range(1, _K):
            tot = tot + vals[i]
        tot = tot + 1e-9
        for i in range(_K):
            g_v[i, pl.ds(off, _L)] = vals[i] / tot
            i_v[i, pl.ds(off, _L)] = ids[i]
        return carry

    lax.fori_loop(0, tpw // _L, group, 0)
    pltpu.sync_copy(g_v, gates_t_hbm.at[:, pl.ds(base, tpw)])
    pltpu.sync_copy(i_v, idx_t_hbm.at[:, pl.ds(base, tpw)])


def _sc_route(probs_t):
    T = probs_t.shape[1]
    tpw = T // _NW
    mesh = plsc.VectorSubcoreMesh(core_axis_name="c", subcore_axis_name="s")
    f = functools.partial(
        pl.kernel,
        mesh=mesh,
        out_type=[
            jax.ShapeDtypeStruct((_K, T), jnp.float32),
            jax.ShapeDtypeStruct((_K, T), jnp.int32),
        ],
        scratch_types=[
            pltpu.VMEM((_E, tpw), jnp.float32),
            pltpu.VMEM((_K, tpw), jnp.float32),
            pltpu.VMEM((_K, tpw), jnp.int32),
        ],
    )(functools.partial(_sc_route_body, tpw))
    return f(probs_t)


_NCHUNKS = 2


@jax.jit
def _router(x2d, W, b2d):
    lg, gs, ix = [], [], []
    for c in range(_NCHUNKS):
        logits_c, probs_t_c = _matmul_probs(x2d, W, b2d, c, _NCHUNKS)
        gates_t_c, idx_t_c = _sc_route(probs_t_c)
        lg.append(logits_c)
        gs.append(gates_t_c.T)
        ix.append(idx_t_c.T)
    if _NCHUNKS == 1:
        return lg[0], gs[0], ix[0]
    logits = jnp.concatenate(lg, axis=0)
    gates = jnp.concatenate(gs, axis=0)
    idx = jnp.concatenate(ix, axis=0)
    return logits, gates, idx


def kernel(x, W, b):
    B, S, D = x.shape
    x2d = x.reshape(B * S, D)
    logits, gates, idx = _router(x2d, W, b.reshape(1, _E))
    return (
        gates.reshape(B, S, _K),
        idx.reshape(B, S, _K),
        logits.reshape(B, S, _E),
    )
